# baseline (speedup 1.0000x reference)
import functools

import jax
import jax.numpy as jnp
from jax import lax
from jax.experimental import pallas as pl
from jax.experimental.pallas import tpu as pltpu
from jax.experimental.pallas import tpu_sc as plsc

N_D = 128
N_P = 100000
L = 16
NC, NS = 2, 16
NW = NC * NS
RPW = N_D // NW
CH = 10000
NCH = N_P // CH
UNROLL = 5
VPC = CH // L
G = 25

_LN2_HI = 0.693359375
_LN2_LO = -2.12194440e-4
_SQRTHF = 0.70710678118654752440
_LOG_POLY = (7.0376836292e-2, -1.1514610310e-1, 1.1676998740e-1,
             -1.2420140846e-1, 1.4249322787e-1, -1.6668057665e-1,
             2.0000714765e-1, -2.4999993993e-1, 3.3333331174e-1)


def _plog(x):
    bits = lax.bitcast_convert_type(x, jnp.int32)
    e = jnp.right_shift(bits, 23) - 126
    m = lax.bitcast_convert_type((bits & 0x007FFFFF) | 0x3F000000,
                                 jnp.float32)
    ef = e.astype(jnp.float32)
    small = m < _SQRTHF
    ef = jnp.where(small, ef - 1.0, ef)
    m = jnp.where(small, m + m, m)
    r = m - 1.0
    z = r * r
    y = jnp.full((L,), _LOG_POLY[0], jnp.float32)
    for coef in _LOG_POLY[1:]:
        y = y * r + coef
    y = y * r * z
    y = y + ef * _LN2_LO
    y = y - 0.5 * z
    return r + y + ef * _LN2_HI


def _take16(x, idx):
    return lax.gather(
        x, idx[:, None],
        lax.GatherDimensionNumbers(offset_dims=(), collapsed_slice_dims=(0,),
                                   start_index_map=(0,)),
        (1,), mode=lax.GatherScatterMode.PROMISE_IN_BOUNDS)


def _allreduce(x, op):
    lane = lax.iota(jnp.int32, L)
    for sh in (1, 2, 4, 8):
        x = op(x, _take16(x, lane ^ sh))
    return x


def _lane0(x):
    return lax.squeeze(lax.slice(x, (0,), (1,)), (0,))


def _make_kernel():
    mesh = plsc.VectorSubcoreMesh(core_axis_name="c", subcore_axis_name="s")

    @functools.partial(
        pl.kernel,
        out_type=[jax.ShapeDtypeStruct((NW * L,), jnp.int32),
                  jax.ShapeDtypeStruct((NW * L,), jnp.float32)],
        mesh=mesh,
        scratch_types=[
            pltpu.VMEM((N_P,), jnp.float32),
            pltpu.VMEM((CH,), jnp.float32),
            pltpu.VMEM((CH,), jnp.float32),
            pltpu.VMEM((L,), jnp.int32),
            pltpu.VMEM((L,), jnp.float32),
            pltpu.VMEM((L,), jnp.float32),
            pltpu.VMEM((L,), jnp.float32),
            pltpu.VMEM((L,), jnp.int32),
            pltpu.VMEM((L,), jnp.float32),
            pltpu.SemaphoreType.DMA,
            pltpu.SemaphoreType.DMA,
            pltpu.SemaphoreType.DMA,
        ],
    )
    def k(params_hbm, noise_hbm, vals_hbm, lps_hbm,
          prow, ubuf0, ubuf1, vstage, lstage,
          nb_buf, tb_buf, ib_buf, thr_buf, psem, nsem0, nsem1):
        wid = lax.axis_index("s") * NC + lax.axis_index("c")
        lane = lax.iota(jnp.int32, L)

        def row_body(rr, row_carry):
            vvals, vlps = row_carry
            row = wid * RPW + rr
            rbase = row * N_P
            pcopy = pltpu.async_copy(
                params_hbm.at[pl.ds(rbase, N_P)], prow, psem)
            pltpu.async_copy(noise_hbm.at[pl.ds(rbase, CH)], ubuf0, nsem0)
            pltpu.async_copy(noise_hbm.at[pl.ds(rbase + CH, CH)],
                             ubuf1, nsem1)
            pcopy.wait()

            def sum_body(i, acc):
                base = i * (UNROLL * L)
                for j in range(UNROLL):
                    acc = acc + prow[pl.ds(base + j * L, L)]
                return acc

            acc = lax.fori_loop(0, N_P // (UNROLL * L), sum_body,
                                jnp.zeros((L,), jnp.float32))
            s_vec = _allreduce(acc, jnp.add)
            cshift = s_vec * 1e-12
            thrv = (1.0 + cshift) * (1.0 + 1e-5)

            nb_buf[...] = jnp.zeros((L,), jnp.float32)
            tb_buf[...] = jnp.ones((L,), jnp.float32)
            ib_buf[...] = jnp.zeros((L,), jnp.int32)
            thr_buf[...] = jnp.full((L,), -1.0, jnp.float32)

            def consume(ch_base, ubuf, uthr_s):
                def grp_body(g, uthr_s):
                    base = g * (G * L)
                    mu = ubuf[pl.ds(base, L)]
                    for j in range(1, G):
                        mu = jnp.maximum(mu, ubuf[pl.ds(base + j * L, L)])
                    mu_s = _lane0(_allreduce(mu, jnp.maximum))

                    @pl.when(mu_s > uthr_s)
                    def _():
                        def vec_body(j, dummy):
                            off = base + j * L
                            u = ubuf[pl.ds(off, L)]
                            u_s = _lane0(_allreduce(u, jnp.maximum))

                            @pl.when(u_s > uthr_s)
                            def _():
                                nb = nb_buf[...]
                                tb = tb_buf[...]
                                ib = ib_buf[...]
                                p = prow[pl.ds(ch_base + off, L)]
                                uc = jnp.minimum(
                                    jnp.maximum(u, 1e-9), 1.0)
                                t = -_plog(uc)
                                num = p + cshift
                                win = (num * tb) > (nb * t)
                                nb_buf[...] = jnp.where(win, num, nb)
                                tb_buf[...] = jnp.where(win, t, tb)
                                ib_buf[...] = jnp.where(
                                    win, ch_base + off + lane, ib)

                            return dummy

                        lax.fori_loop(0, G, vec_body, 0)
                        gr = _allreduce(nb_buf[...] / tb_buf[...],
                                        jnp.maximum)
                        thr_buf[...] = 1.0 - thrv / gr - 1e-6

                    return _lane0(thr_buf[...])

                return lax.fori_loop(0, VPC // G, grp_body, uthr_s)

            def pair_body(pi, uthr_s):
                ch0 = 2 * pi
                pltpu.make_async_copy(
                    noise_hbm.at[pl.ds(rbase + ch0 * CH, CH)],
                    ubuf0, nsem0).wait()
                uthr_s = consume(ch0 * CH, ubuf0, uthr_s)

                @pl.when(pi + 1 < NCH // 2)
                def _():
                    pltpu.async_copy(
                        noise_hbm.at[pl.ds(rbase + (ch0 + 2) * CH, CH)],
                        ubuf0, nsem0)

                pltpu.make_async_copy(
                    noise_hbm.at[pl.ds(rbase + (ch0 + 1) * CH, CH)],
                    ubuf1, nsem1).wait()
                uthr_s = consume((ch0 + 1) * CH, ubuf1, uthr_s)

                @pl.when(pi + 1 < NCH // 2)
                def _():
                    pltpu.async_copy(
                        noise_hbm.at[pl.ds(rbase + (ch0 + 3) * CH, CH)],
                        ubuf1, nsem1)

                return uthr_s

            lax.fori_loop(0, NCH // 2, pair_body, jnp.float32(-1.0))
            nb = nb_buf[...]
            tb = tb_buf[...]
            ib = ib_buf[...]

            sb = nb / tb
            mx = _allreduce(sb, jnp.maximum)
            eq = sb == mx
            iw = _allreduce(jnp.where(eq, ib, -1), jnp.maximum)
            numw = _allreduce(jnp.where(eq, nb, 0.0), jnp.maximum)
            lp = _plog(numw / s_vec)
            vvals = jnp.where(lane == rr, iw, vvals)
            vlps = jnp.where(lane == rr, lp, vlps)
            return vvals, vlps

        vvals, vlps = lax.fori_loop(
            0, RPW, row_body,
            (jnp.zeros((L,), jnp.int32), jnp.zeros((L,), jnp.float32)))

        vstage[...] = vvals
        lstage[...] = vlps
        pltpu.sync_copy(vstage, vals_hbm.at[pl.ds(wid * L, L)])
        pltpu.sync_copy(lstage, lps_hbm.at[pl.ds(wid * L, L)])

    return k


_sc_kernel = _make_kernel()


def kernel(params, noise):
    vals2, lps2 = _sc_kernel(params.reshape(-1), noise.reshape(-1))
    values = vals2.reshape(NW, L)[:, :RPW].reshape(N_D)
    log_probs = lps2.reshape(NW, L)[:, :RPW].reshape(N_D)
    return values, log_probs


# speedup vs baseline: 1.2211x; 1.2211x over previous
import functools

import jax
import jax.numpy as jnp
from jax import lax
from jax.experimental import pallas as pl
from jax.experimental.pallas import tpu as pltpu
from jax.experimental.pallas import tpu_sc as plsc

N_D = 128
N_P = 100000
L = 16
NC, NS = 2, 16
NW = NC * NS
RPW = N_D // NW
CH = 10000
NCH = N_P // CH
UNROLL = 5
VPC = CH // L
G = 25

_LN2_HI = 0.693359375
_LN2_LO = -2.12194440e-4
_SQRTHF = 0.70710678118654752440
_LOG_POLY = (7.0376836292e-2, -1.1514610310e-1, 1.1676998740e-1,
             -1.2420140846e-1, 1.4249322787e-1, -1.6668057665e-1,
             2.0000714765e-1, -2.4999993993e-1, 3.3333331174e-1)


def _plog(x):
    bits = lax.bitcast_convert_type(x, jnp.int32)
    e = jnp.right_shift(bits, 23) - 126
    m = lax.bitcast_convert_type((bits & 0x007FFFFF) | 0x3F000000,
                                 jnp.float32)
    ef = e.astype(jnp.float32)
    small = m < _SQRTHF
    ef = jnp.where(small, ef - 1.0, ef)
    m = jnp.where(small, m + m, m)
    r = m - 1.0
    z = r * r
    y = jnp.full((L,), _LOG_POLY[0], jnp.float32)
    for coef in _LOG_POLY[1:]:
        y = y * r + coef
    y = y * r * z
    y = y + ef * _LN2_LO
    y = y - 0.5 * z
    return r + y + ef * _LN2_HI


def _take16(x, idx):
    return lax.gather(
        x, idx[:, None],
        lax.GatherDimensionNumbers(offset_dims=(), collapsed_slice_dims=(0,),
                                   start_index_map=(0,)),
        (1,), mode=lax.GatherScatterMode.PROMISE_IN_BOUNDS)


def _allreduce(x, op):
    lane = lax.iota(jnp.int32, L)
    for sh in (1, 2, 4, 8):
        x = op(x, _take16(x, lane ^ sh))
    return x


def _lane0(x):
    return lax.squeeze(lax.slice(x, (0,), (1,)), (0,))


NGR = N_P // (G * L)
GPC = VPC // G


def _make_kernel():
    mesh = plsc.VectorSubcoreMesh(core_axis_name="c", subcore_axis_name="s")

    @functools.partial(
        pl.kernel,
        out_type=[jax.ShapeDtypeStruct((NW * L,), jnp.int32),
                  jax.ShapeDtypeStruct((NW * L,), jnp.float32)],
        mesh=mesh,
        scratch_types=[
            pltpu.VMEM((CH,), jnp.float32),
            pltpu.VMEM((CH,), jnp.float32),
            pltpu.VMEM((CH,), jnp.float32),
            pltpu.VMEM((CH,), jnp.float32),
            pltpu.VMEM((NGR * L,), jnp.float32),
            pltpu.VMEM((G * L,), jnp.float32),
            pltpu.VMEM((G * L,), jnp.float32),
            pltpu.VMEM((L,), jnp.int32),
            pltpu.VMEM((L,), jnp.float32),
            pltpu.VMEM((L,), jnp.float32),
            pltpu.VMEM((L,), jnp.float32),
            pltpu.VMEM((L,), jnp.int32),
            pltpu.VMEM((L,), jnp.float32),
            pltpu.SemaphoreType.DMA,
            pltpu.SemaphoreType.DMA,
            pltpu.SemaphoreType.DMA,
            pltpu.SemaphoreType.DMA,
            pltpu.SemaphoreType.DMA,
            pltpu.SemaphoreType.DMA,
        ],
    )
    def k(params_hbm, noise_hbm, vals_hbm, lps_hbm,
          pbuf0, pbuf1, ubuf0, ubuf1, gmu, pgrp, ugrp, vstage, lstage,
          nb_buf, tb_buf, ib_buf, thr_buf,
          psem0, psem1, nsem0, nsem1, gsem0, gsem1):
        wid = lax.axis_index("s") * NC + lax.axis_index("c")
        lane = lax.iota(jnp.int32, L)

        def issue_pair(rb, ch0):
            pltpu.async_copy(params_hbm.at[pl.ds(rb + ch0 * CH, CH)],
                             pbuf0, psem0)
            pltpu.async_copy(noise_hbm.at[pl.ds(rb + ch0 * CH, CH)],
                             ubuf0, nsem0)
            pltpu.async_copy(params_hbm.at[pl.ds(rb + (ch0 + 1) * CH, CH)],
                             pbuf1, psem1)
            pltpu.async_copy(noise_hbm.at[pl.ds(rb + (ch0 + 1) * CH, CH)],
                             ubuf1, nsem1)

        issue_pair(wid * RPW * N_P, 0)

        def row_body(rr, row_carry):
            vvals, vlps = row_carry
            row = wid * RPW + rr
            rbase = row * N_P

            def consume(ci, pbuf, ubuf, acc):
                def grp_body(g, acc):
                    base = g * (G * L)
                    mu = ubuf[pl.ds(base, L)]
                    acc = acc + pbuf[pl.ds(base, L)]
                    for j in range(1, G):
                        off = base + j * L
                        acc = acc + pbuf[pl.ds(off, L)]
                        mu = jnp.maximum(mu, ubuf[pl.ds(off, L)])
                    gmu[pl.ds((ci * GPC + g) * L, L)] = mu
                    return acc

                return lax.fori_loop(0, GPC, grp_body, acc)

            def pair_body(pi, acc):
                ch0 = 2 * pi
                pltpu.make_async_copy(
                    params_hbm.at[pl.ds(rbase + ch0 * CH, CH)],
                    pbuf0, psem0).wait()
                pltpu.make_async_copy(
                    noise_hbm.at[pl.ds(rbase + ch0 * CH, CH)],
                    ubuf0, nsem0).wait()
                acc = consume(ch0, pbuf0, ubuf0, acc)

                @pl.when(pi + 1 < NCH // 2)
                def _():
                    pltpu.async_copy(
                        params_hbm.at[pl.ds(rbase + (ch0 + 2) * CH, CH)],
                        pbuf0, psem0)
                    pltpu.async_copy(
                        noise_hbm.at[pl.ds(rbase + (ch0 + 2) * CH, CH)],
                        ubuf0, nsem0)

                pltpu.make_async_copy(
                    params_hbm.at[pl.ds(rbase + (ch0 + 1) * CH, CH)],
                    pbuf1, psem1).wait()
                pltpu.make_async_copy(
                    noise_hbm.at[pl.ds(rbase + (ch0 + 1) * CH, CH)],
                    ubuf1, nsem1).wait()
                acc = consume(ch0 + 1, pbuf1, ubuf1, acc)

                @pl.when(pi + 1 < NCH // 2)
                def _():
                    pltpu.async_copy(
                        params_hbm.at[pl.ds(rbase + (ch0 + 3) * CH, CH)],
                        pbuf1, psem1)
                    pltpu.async_copy(
                        noise_hbm.at[pl.ds(rbase + (ch0 + 3) * CH, CH)],
                        ubuf1, nsem1)

                return acc

            acc = lax.fori_loop(0, NCH // 2, pair_body,
                                jnp.zeros((L,), jnp.float32))

            @pl.when(rr + 1 < RPW)
            def _():
                issue_pair(rbase + N_P, 0)

            s_vec = _allreduce(acc, jnp.add)
            cshift = s_vec * 1e-12
            thrv = (1.0 + cshift) * (1.0 + 1e-5)

            nb_buf[...] = jnp.zeros((L,), jnp.float32)
            tb_buf[...] = jnp.ones((L,), jnp.float32)
            ib_buf[...] = jnp.zeros((L,), jnp.int32)

            def eval_group(gs):
                ebase = gs * (G * L)
                cp = pltpu.async_copy(
                    params_hbm.at[pl.ds(rbase + ebase, G * L)],
                    pgrp, gsem0)
                cu = pltpu.async_copy(
                    noise_hbm.at[pl.ds(rbase + ebase, G * L)],
                    ugrp, gsem1)
                cp.wait()
                cu.wait()

                def vec_body(j, dummy):
                    off = j * L
                    u = ugrp[pl.ds(off, L)]
                    p = pgrp[pl.ds(off, L)]
                    nb = nb_buf[...]
                    tb = tb_buf[...]
                    ib = ib_buf[...]
                    uc = jnp.minimum(jnp.maximum(u, 1e-9), 1.0)
                    t = -_plog(uc)
                    num = p + cshift
                    win = (num * tb) > (nb * t)
                    nb_buf[...] = jnp.where(win, num, nb)
                    tb_buf[...] = jnp.where(win, t, tb)
                    ib_buf[...] = jnp.where(win, ebase + off + lane, ib)
                    return dummy

                lax.fori_loop(0, G, vec_body, 0)
                gr = _allreduce(nb_buf[...] / tb_buf[...], jnp.maximum)
                thr_buf[...] = 1.0 - thrv / gr - 1e-6

            def fmax_body(g, carry):
                runmax, gidx = carry
                m = gmu[pl.ds(g * L, L)]
                w = m > runmax
                return (jnp.where(w, m, runmax), jnp.where(w, g, gidx))

            runmax, gidx = lax.fori_loop(
                0, NGR, fmax_body,
                (jnp.full((L,), -1.0, jnp.float32),
                 jnp.zeros((L,), jnp.int32)))
            mvec = _allreduce(runmax, jnp.maximum)
            gstar = _lane0(_allreduce(
                jnp.where(runmax == mvec, gidx, -1), jnp.maximum))
            eval_group(gstar)

            def scan_body(g, uthr_s):
                m = gmu[pl.ds(g * L, L)]
                mu_s = _lane0(_allreduce(m, jnp.maximum))

                @pl.when(jnp.logical_and(mu_s > uthr_s, g != gstar))
                def _():
                    eval_group(g)

                return _lane0(thr_buf[...])

            lax.fori_loop(0, NGR, scan_body, _lane0(thr_buf[...]))
            nb = nb_buf[...]
            tb = tb_buf[...]
            ib = ib_buf[...]

            sb = nb / tb
            mx = _allreduce(sb, jnp.maximum)
            eq = sb == mx
            iw = _allreduce(jnp.where(eq, ib, -1), jnp.maximum)
            numw = _allreduce(jnp.where(eq, nb, 0.0), jnp.maximum)
            lp = _plog(numw / s_vec)
            vvals = jnp.where(lane == rr, iw, vvals)
            vlps = jnp.where(lane == rr, lp, vlps)
            return vvals, vlps

        vvals, vlps = lax.fori_loop(
            0, RPW, row_body,
            (jnp.zeros((L,), jnp.int32), jnp.zeros((L,), jnp.float32)))

        vstage[...] = vvals
        lstage[...] = vlps
        pltpu.sync_copy(vstage, vals_hbm.at[pl.ds(wid * L, L)])
        pltpu.sync_copy(lstage, lps_hbm.at[pl.ds(wid * L, L)])

    return k


_sc_kernel = _make_kernel()


def kernel(params, noise):
    vals2, lps2 = _sc_kernel(params.reshape(-1), noise.reshape(-1))
    values = vals2.reshape(NW, L)[:, :RPW].reshape(N_D)
    log_probs = lps2.reshape(NW, L)[:, :RPW].reshape(N_D)
    return values, log_probs
